# Initial kernel scaffold; baseline (speedup 1.0000x reference)
import jax
import jax.numpy as jnp
from jax.experimental import pallas as pl


def kernel(x, edge_index, W_proj, b_proj, W_gcn, b_gcn):
    raise NotImplementedError("write your pallas kernel here")


# speedup vs baseline: 30.4090x; 30.4090x over previous
import functools

import jax
import jax.numpy as jnp
from jax import lax
from jax.experimental import pallas as pl
from jax.experimental.pallas import tpu as pltpu
from jax.experimental.pallas import tpu_sc as plsc

N = 10000
D = 128
E = 320000

NC = 2
NS = 16
K = 128

N_PAD = 10240
E_PAD = 327680
ROWS = E_PAD // K
ROWS_PER_TILE = ROWS // (NC * NS)
N_PER_TILE = N_PAD // NS


def _make_deg_kernel():
  mesh = plsc.VectorSubcoreMesh(core_axis_name="c", subcore_axis_name="s",
                                num_cores=NC, num_subcores=NS)

  @functools.partial(
      pl.kernel,
      out_type=jax.ShapeDtypeStruct((NC, N_PAD), jnp.float32),
      mesh=mesh,
      scratch_types=[
          pltpu.VMEM((ROWS_PER_TILE, K), jnp.int32),
          pltpu.VMEM((K,), jnp.float32),
          pltpu.VMEM((N_PER_TILE,), jnp.float32),
          pltpu.VMEM_SHARED((N_PAD,), jnp.float32),
      ],
  )
  def deg_kernel(dst_hbm, out_hbm, idx_v, ones_v, zeros_v, deg_sh):
    cid = lax.axis_index("c")
    sid = lax.axis_index("s")

    @pl.loop(0, N_PER_TILE // 16)
    def _zero(i):
      zeros_v[pl.ds(i * 16, 16)] = jnp.zeros((16,), jnp.float32)

    @pl.loop(0, K // 16)
    def _one(i):
      ones_v[pl.ds(i * 16, 16)] = jnp.ones((16,), jnp.float32)

    pltpu.sync_copy(zeros_v, deg_sh.at[pl.ds(sid * N_PER_TILE, N_PER_TILE)])
    plsc.subcore_barrier()

    row_base = (cid * NS + sid) * ROWS_PER_TILE
    pltpu.sync_copy(dst_hbm.at[pl.ds(row_base, ROWS_PER_TILE)], idx_v)

    @pl.loop(0, ROWS_PER_TILE)
    def _scatter(j):
      pltpu.sync_copy(ones_v, deg_sh.at[idx_v.at[j]], add=True)

    plsc.subcore_barrier()
    pltpu.sync_copy(deg_sh.at[pl.ds(sid * N_PER_TILE, N_PER_TILE)],
                    out_hbm.at[cid, pl.ds(sid * N_PER_TILE, N_PER_TILE)])

  return deg_kernel


def _make_agg_kernel():
  mesh = plsc.VectorSubcoreMesh(core_axis_name="c", subcore_axis_name="s",
                                num_cores=NC, num_subcores=NS)

  @functools.partial(
      pl.kernel,
      out_type=jax.ShapeDtypeStruct((NC, N_PAD, D), jnp.float32),
      mesh=mesh,
      scratch_types=[
          pltpu.VMEM((ROWS_PER_TILE, K), jnp.int32),
          pltpu.VMEM((ROWS_PER_TILE, K), jnp.int32),
          pltpu.VMEM((K, D), jnp.float32),
          pltpu.VMEM((16, D), jnp.float32),
          pltpu.VMEM_SHARED((N_PAD, D), jnp.float32),
          pltpu.SemaphoreType.DMA,
      ],
  )
  def agg_kernel(src_hbm, dst_hbm, mp_hbm, out_hbm,
                 sidx_v, didx_v, rows_v, zeros_v, acc_sh, gsem):
    cid = lax.axis_index("c")
    sid = lax.axis_index("s")

    @pl.loop(0, 16)
    def _zero(r):
      for c in range(D // 16):
        zeros_v[r, pl.ds(c * 16, 16)] = jnp.zeros((16,), jnp.float32)

    @pl.loop(0, N_PER_TILE // 16)
    def _zacc(k):
      pltpu.sync_copy(zeros_v, acc_sh.at[pl.ds(sid * N_PER_TILE + k * 16, 16)])

    plsc.subcore_barrier()

    row_base = (cid * NS + sid) * ROWS_PER_TILE
    pltpu.sync_copy(src_hbm.at[pl.ds(row_base, ROWS_PER_TILE)], sidx_v)
    pltpu.sync_copy(dst_hbm.at[pl.ds(row_base, ROWS_PER_TILE)], didx_v)

    @pl.loop(0, ROWS_PER_TILE)
    def _edge(j):
      pltpu.async_copy(mp_hbm.at[sidx_v.at[j]], rows_v, gsem).wait()
      pltpu.sync_copy(rows_v, acc_sh.at[didx_v.at[j]], add=True)

    plsc.subcore_barrier()

    @pl.loop(0, N_PER_TILE // K)
    def _out(k):
      r = sid * N_PER_TILE + k * K
      pltpu.sync_copy(acc_sh.at[pl.ds(r, K)], out_hbm.at[cid, pl.ds(r, K)])

  return agg_kernel


def _dense_body(x_ref, wp_ref, bp_ref, wg_ref, dega_ref, degb_ref,
                mp_ref, dis_ref):
  h = jnp.maximum(
      jnp.dot(x_ref[...], wp_ref[...], preferred_element_type=jnp.float32)
      + bp_ref[...], 0.0)
  m = jnp.dot(h, wg_ref[...], preferred_element_type=jnp.float32)
  deg = dega_ref[...] + degb_ref[...] + 1.0
  dis = lax.rsqrt(deg)
  dis_ref[...] = dis
  mp_ref[...] = dis * m


def _combine_body(acca_ref, accb_ref, mp_ref, dis_ref, bg_ref, out_ref):
  out_ref[...] = (dis_ref[...]
                  * (acca_ref[...] + accb_ref[...] + mp_ref[...])
                  + bg_ref[...])


_RB = 1000


def _dense_call(x, W_proj, b_proj, W_gcn, degA, degB):
  grid = N // _RB
  row_spec = pl.BlockSpec((_RB, D), lambda i: (i, 0))
  col_spec = pl.BlockSpec((_RB, 1), lambda i: (i, 0))
  full_spec = pl.BlockSpec((D, D), lambda i: (0, 0))
  bias_spec = pl.BlockSpec((1, D), lambda i: (0, 0))
  return pl.pallas_call(
      _dense_body,
      grid=(grid,),
      in_specs=[row_spec, full_spec, bias_spec, full_spec, col_spec, col_spec],
      out_specs=[row_spec, col_spec],
      out_shape=[jax.ShapeDtypeStruct((N, D), jnp.float32),
                 jax.ShapeDtypeStruct((N, 1), jnp.float32)],
  )(x, W_proj, b_proj.reshape(1, D), W_gcn, degA, degB)


def _combine_call(accA, accB, mp, dis, b_gcn):
  grid = N // _RB
  row_spec = pl.BlockSpec((_RB, D), lambda i: (i, 0))
  col_spec = pl.BlockSpec((_RB, 1), lambda i: (i, 0))
  bias_spec = pl.BlockSpec((1, D), lambda i: (0, 0))
  return pl.pallas_call(
      _combine_body,
      grid=(grid,),
      in_specs=[row_spec, row_spec, row_spec, col_spec, bias_spec],
      out_specs=row_spec,
      out_shape=jax.ShapeDtypeStruct((N, D), jnp.float32),
  )(accA, accB, mp, dis, b_gcn.reshape(1, D))


def kernel(x, edge_index, W_proj, b_proj, W_gcn, b_gcn):
  ei = edge_index.astype(jnp.int32)
  src, dst = ei[0], ei[1]

  pad = E_PAD - E
  pad_iota = jnp.arange(pad, dtype=jnp.int32)
  src_p = jnp.concatenate([src, pad_iota % N]).reshape(ROWS, K)
  dst_p = jnp.concatenate([dst, N + pad_iota % (N_PAD - N)]).reshape(ROWS, K)

  deg_p = _make_deg_kernel()(dst_p)
  degA = deg_p[0, :N].reshape(N, 1)
  degB = deg_p[1, :N].reshape(N, 1)

  mp, dis = _dense_call(x, W_proj, b_proj, W_gcn, degA, degB)

  acc_p = _make_agg_kernel()(src_p, dst_p, mp)
  accA = acc_p[0, :N]
  accB = acc_p[1, :N]

  return _combine_call(accA, accB, mp, dis, b_gcn)


# speedup vs baseline: 36.7129x; 1.2073x over previous
import functools

import jax
import jax.numpy as jnp
from jax import lax
from jax.experimental import pallas as pl
from jax.experimental.pallas import tpu as pltpu
from jax.experimental.pallas import tpu_sc as plsc

N = 10000
D = 128
E = 320000

NC = 2
NS = 16
K = 128

N_PAD = 10240
E_PAD = 327680
ROWS = E_PAD // K
ROWS_PER_TILE = ROWS // (NC * NS)
N_PER_TILE = N_PAD // NS

KC = 64
CROWS = E_PAD // KC
CPT = CROWS // (NC * NS)


def _make_deg_kernel():
  mesh = plsc.VectorSubcoreMesh(core_axis_name="c", subcore_axis_name="s",
                                num_cores=NC, num_subcores=NS)

  @functools.partial(
      pl.kernel,
      out_type=jax.ShapeDtypeStruct((NC, N_PAD), jnp.float32),
      mesh=mesh,
      scratch_types=[
          pltpu.VMEM((ROWS_PER_TILE, K), jnp.int32),
          pltpu.VMEM((K,), jnp.float32),
          pltpu.VMEM((N_PER_TILE,), jnp.float32),
          pltpu.VMEM_SHARED((N_PAD,), jnp.float32),
      ],
  )
  def deg_kernel(dst_hbm, out_hbm, idx_v, ones_v, zeros_v, deg_sh):
    cid = lax.axis_index("c")
    sid = lax.axis_index("s")

    @pl.loop(0, N_PER_TILE // 16)
    def _zero(i):
      zeros_v[pl.ds(i * 16, 16)] = jnp.zeros((16,), jnp.float32)

    @pl.loop(0, K // 16)
    def _one(i):
      ones_v[pl.ds(i * 16, 16)] = jnp.ones((16,), jnp.float32)

    pltpu.sync_copy(zeros_v, deg_sh.at[pl.ds(sid * N_PER_TILE, N_PER_TILE)])
    plsc.subcore_barrier()

    row_base = (cid * NS + sid) * ROWS_PER_TILE
    pltpu.sync_copy(dst_hbm.at[pl.ds(row_base, ROWS_PER_TILE)], idx_v)

    @pl.loop(0, ROWS_PER_TILE)
    def _scatter(j):
      pltpu.sync_copy(ones_v, deg_sh.at[idx_v.at[j]], add=True)

    plsc.subcore_barrier()
    pltpu.sync_copy(deg_sh.at[pl.ds(sid * N_PER_TILE, N_PER_TILE)],
                    out_hbm.at[cid, pl.ds(sid * N_PER_TILE, N_PER_TILE)])

  return deg_kernel


def _make_agg_kernel():
  mesh = plsc.VectorSubcoreMesh(core_axis_name="c", subcore_axis_name="s",
                                num_cores=NC, num_subcores=NS)

  @functools.partial(
      pl.kernel,
      out_type=jax.ShapeDtypeStruct((NC, N_PAD, D), jnp.float32),
      mesh=mesh,
      scratch_types=[
          pltpu.VMEM((CPT // 2, KC), jnp.int32),
          pltpu.VMEM((CPT // 2, KC), jnp.int32),
          pltpu.VMEM((KC, D), jnp.float32),
          pltpu.VMEM((KC, D), jnp.float32),
          pltpu.VMEM((16, D), jnp.float32),
          pltpu.VMEM_SHARED((N_PAD, D), jnp.float32),
          pltpu.SemaphoreType.DMA,
          pltpu.SemaphoreType.DMA,
      ],
  )
  def agg_kernel(src_hbm, dst_hbm, mp_hbm, out_hbm,
                 sidx_v, didx_v, rows0_v, rows1_v, zeros_v, acc_sh,
                 gsem0, gsem1):
    cid = lax.axis_index("c")
    sid = lax.axis_index("s")

    @pl.loop(0, 16)
    def _zero(r):
      for c in range(D // 16):
        zeros_v[r, pl.ds(c * 16, 16)] = jnp.zeros((16,), jnp.float32)

    @pl.loop(0, N_PER_TILE // 16)
    def _zacc(k):
      pltpu.sync_copy(zeros_v, acc_sh.at[pl.ds(sid * N_PER_TILE + k * 16, 16)])

    plsc.subcore_barrier()

    row_base = (cid * NS + sid) * CPT
    half_rows = CPT // 2
    npairs = half_rows // 2

    for half in range(2):
      base = row_base + half * half_rows
      pltpu.sync_copy(src_hbm.at[pl.ds(base, half_rows)], sidx_v)
      pltpu.sync_copy(dst_hbm.at[pl.ds(base, half_rows)], didx_v)

      pltpu.async_copy(mp_hbm.at[sidx_v.at[0]], rows0_v, gsem0)
      pltpu.async_copy(mp_hbm.at[sidx_v.at[1]], rows1_v, gsem1)

      @pl.loop(0, npairs)
      def _pair(p):
        c0 = 2 * p
        pltpu.make_async_copy(mp_hbm.at[sidx_v.at[0]], rows0_v, gsem0).wait()
        pltpu.sync_copy(rows0_v, acc_sh.at[didx_v.at[c0]], add=True)

        @pl.when(p < npairs - 1)
        def _g0():
          pltpu.async_copy(mp_hbm.at[sidx_v.at[c0 + 2]], rows0_v, gsem0)

        pltpu.make_async_copy(mp_hbm.at[sidx_v.at[1]], rows1_v, gsem1).wait()
        pltpu.sync_copy(rows1_v, acc_sh.at[didx_v.at[c0 + 1]], add=True)

        @pl.when(p < npairs - 1)
        def _g1():
          pltpu.async_copy(mp_hbm.at[sidx_v.at[c0 + 3]], rows1_v, gsem1)

    plsc.subcore_barrier()

    @pl.loop(0, N_PER_TILE // K)
    def _out(k):
      r = sid * N_PER_TILE + k * K
      pltpu.sync_copy(acc_sh.at[pl.ds(r, K)], out_hbm.at[cid, pl.ds(r, K)])

  return agg_kernel


def _dense_body(x_ref, wp_ref, bp_ref, wg_ref, dega_ref, degb_ref,
                mp_ref, dis_ref):
  h = jnp.maximum(
      jnp.dot(x_ref[...], wp_ref[...], preferred_element_type=jnp.float32)
      + bp_ref[...], 0.0)
  m = jnp.dot(h, wg_ref[...], preferred_element_type=jnp.float32)
  deg = dega_ref[...] + degb_ref[...] + 1.0
  dis = lax.rsqrt(deg)
  dis_ref[...] = dis
  mp_ref[...] = dis * m


def _combine_body(acca_ref, accb_ref, mp_ref, dis_ref, bg_ref, out_ref):
  out_ref[...] = (dis_ref[...]
                  * (acca_ref[...] + accb_ref[...] + mp_ref[...])
                  + bg_ref[...])


_RB = 1000


def _dense_call(x, W_proj, b_proj, W_gcn, degA, degB):
  grid = N // _RB
  row_spec = pl.BlockSpec((_RB, D), lambda i: (i, 0))
  col_spec = pl.BlockSpec((_RB, 1), lambda i: (i, 0))
  full_spec = pl.BlockSpec((D, D), lambda i: (0, 0))
  bias_spec = pl.BlockSpec((1, D), lambda i: (0, 0))
  return pl.pallas_call(
      _dense_body,
      grid=(grid,),
      in_specs=[row_spec, full_spec, bias_spec, full_spec, col_spec, col_spec],
      out_specs=[row_spec, col_spec],
      out_shape=[jax.ShapeDtypeStruct((N, D), jnp.float32),
                 jax.ShapeDtypeStruct((N, 1), jnp.float32)],
  )(x, W_proj, b_proj.reshape(1, D), W_gcn, degA, degB)


def _combine_call(accA, accB, mp, dis, b_gcn):
  grid = N // _RB
  row_spec = pl.BlockSpec((_RB, D), lambda i: (i, 0))
  col_spec = pl.BlockSpec((_RB, 1), lambda i: (i, 0))
  bias_spec = pl.BlockSpec((1, D), lambda i: (0, 0))
  return pl.pallas_call(
      _combine_body,
      grid=(grid,),
      in_specs=[row_spec, row_spec, row_spec, col_spec, bias_spec],
      out_specs=row_spec,
      out_shape=jax.ShapeDtypeStruct((N, D), jnp.float32),
  )(accA, accB, mp, dis, b_gcn.reshape(1, D))


def kernel(x, edge_index, W_proj, b_proj, W_gcn, b_gcn):
  ei = edge_index.astype(jnp.int32)
  src, dst = ei[0], ei[1]

  pad = E_PAD - E
  pad_iota = jnp.arange(pad, dtype=jnp.int32)
  src_flat = jnp.concatenate([src, pad_iota % N])
  dst_flat = jnp.concatenate([dst, N + pad_iota % (N_PAD - N)])
  dst_p = dst_flat.reshape(ROWS, K)
  src_c = src_flat.reshape(CROWS, KC)
  dst_c = dst_flat.reshape(CROWS, KC)

  deg_p = _make_deg_kernel()(dst_p)
  degA = deg_p[0, :N].reshape(N, 1)
  degB = deg_p[1, :N].reshape(N, 1)

  mp, dis = _dense_call(x, W_proj, b_proj, W_gcn, degA, degB)

  acc_p = _make_agg_kernel()(src_c, dst_c, mp)
  accA = acc_p[0, :N]
  accB = acc_p[1, :N]

  return _combine_call(accA, accB, mp, dis, b_gcn)


# speedup vs baseline: 37.2392x; 1.0143x over previous
import functools

import jax
import jax.numpy as jnp
from jax import lax
from jax.experimental import pallas as pl
from jax.experimental.pallas import tpu as pltpu
from jax.experimental.pallas import tpu_sc as plsc

N = 10000
D = 128
E = 320000

NC = 2
NS = 16

N_PAD = 10240
E_PAD = 327680
N_PER_TILE = N_PAD // NS

KC = 64
CROWS = E_PAD // KC
CPT = CROWS // (NC * NS)
K = 128
ROWS = E_PAD // K
ROWS_PER_TILE = ROWS // (NC * NS)


def _make_deg_kernel():
  mesh = plsc.VectorSubcoreMesh(core_axis_name="c", subcore_axis_name="s",
                                num_cores=NC, num_subcores=NS)

  @functools.partial(
      pl.kernel,
      out_type=[jax.ShapeDtypeStruct((N_PAD,), jnp.float32),
                jax.ShapeDtypeStruct((N_PAD,), jnp.float32)],
      mesh=mesh,
      scratch_types=[
          pltpu.VMEM((ROWS_PER_TILE, K), jnp.int32),
          pltpu.VMEM((K,), jnp.float32),
          pltpu.VMEM((N_PER_TILE,), jnp.float32),
          pltpu.VMEM_SHARED((N_PAD,), jnp.float32),
      ],
  )
  def deg_kernel(dst_hbm, outa_hbm, outb_hbm, idx_v, ones_v, zeros_v, deg_sh):
    cid = lax.axis_index("c")
    sid = lax.axis_index("s")

    @pl.loop(0, N_PER_TILE // 16)
    def _zero(i):
      zeros_v[pl.ds(i * 16, 16)] = jnp.zeros((16,), jnp.float32)

    @pl.loop(0, K // 16)
    def _one(i):
      ones_v[pl.ds(i * 16, 16)] = jnp.ones((16,), jnp.float32)

    pltpu.sync_copy(zeros_v, deg_sh.at[pl.ds(sid * N_PER_TILE, N_PER_TILE)])
    plsc.subcore_barrier()

    row_base = (cid * NS + sid) * ROWS_PER_TILE
    pltpu.sync_copy(dst_hbm.at[pl.ds(row_base, ROWS_PER_TILE)], idx_v)

    @pl.loop(0, ROWS_PER_TILE)
    def _scatter(j):
      pltpu.sync_copy(ones_v, deg_sh.at[idx_v.at[j]], add=True)

    plsc.subcore_barrier()

    @pl.when(cid == 0)
    def _outa():
      pltpu.sync_copy(deg_sh.at[pl.ds(sid * N_PER_TILE, N_PER_TILE)],
                      outa_hbm.at[pl.ds(sid * N_PER_TILE, N_PER_TILE)])

    @pl.when(cid == 1)
    def _outb():
      pltpu.sync_copy(deg_sh.at[pl.ds(sid * N_PER_TILE, N_PER_TILE)],
                      outb_hbm.at[pl.ds(sid * N_PER_TILE, N_PER_TILE)])

  return deg_kernel


def _make_agg_kernel():
  mesh = plsc.VectorSubcoreMesh(core_axis_name="c", subcore_axis_name="s",
                                num_cores=NC, num_subcores=NS)

  @functools.partial(
      pl.kernel,
      out_type=jax.ShapeDtypeStruct((NC, N_PAD, D), jnp.float32),
      mesh=mesh,
      scratch_types=[
          pltpu.VMEM((CPT // 2, KC), jnp.int32),
          pltpu.VMEM((CPT // 2, KC), jnp.int32),
          pltpu.VMEM((KC, D), jnp.float32),
          pltpu.VMEM((KC, D), jnp.float32),
          pltpu.VMEM((16, D), jnp.float32),
          pltpu.VMEM_SHARED((N_PAD, D), jnp.float32),
          pltpu.SemaphoreType.DMA,
          pltpu.SemaphoreType.DMA,
      ],
  )
  def agg_kernel(src_hbm, dst_hbm, mp_hbm, out_hbm,
                 sidx_v, didx_v, rows0_v, rows1_v, zeros_v, acc_sh,
                 gsem0, gsem1):
    cid = lax.axis_index("c")
    sid = lax.axis_index("s")

    @pl.loop(0, 16)
    def _zero(r):
      for c in range(D // 16):
        zeros_v[r, pl.ds(c * 16, 16)] = jnp.zeros((16,), jnp.float32)

    @pl.loop(0, N_PER_TILE // 16)
    def _zacc(k):
      pltpu.sync_copy(zeros_v, acc_sh.at[pl.ds(sid * N_PER_TILE + k * 16, 16)])

    plsc.subcore_barrier()

    row_base = (cid * NS + sid) * CPT
    half_rows = CPT // 2
    npairs = half_rows // 2

    for half in range(2):
      base = row_base + half * half_rows
      pltpu.sync_copy(src_hbm.at[pl.ds(base, half_rows)], sidx_v)
      pltpu.sync_copy(dst_hbm.at[pl.ds(base, half_rows)], didx_v)

      pltpu.async_copy(mp_hbm.at[sidx_v.at[0]], rows0_v, gsem0)
      pltpu.async_copy(mp_hbm.at[sidx_v.at[1]], rows1_v, gsem1)

      @pl.loop(0, npairs)
      def _pair(p):
        c0 = 2 * p
        pltpu.make_async_copy(mp_hbm.at[sidx_v.at[0]], rows0_v, gsem0).wait()
        pltpu.sync_copy(rows0_v, acc_sh.at[didx_v.at[c0]], add=True)

        @pl.when(p < npairs - 1)
        def _g0():
          pltpu.async_copy(mp_hbm.at[sidx_v.at[c0 + 2]], rows0_v, gsem0)

        pltpu.make_async_copy(mp_hbm.at[sidx_v.at[1]], rows1_v, gsem1).wait()
        pltpu.sync_copy(rows1_v, acc_sh.at[didx_v.at[c0 + 1]], add=True)

        @pl.when(p < npairs - 1)
        def _g1():
          pltpu.async_copy(mp_hbm.at[sidx_v.at[c0 + 3]], rows1_v, gsem1)

    plsc.subcore_barrier()

    @pl.loop(0, N_PER_TILE // K)
    def _out(k):
      r = sid * N_PER_TILE + k * K
      pltpu.sync_copy(acc_sh.at[pl.ds(r, K)], out_hbm.at[cid, pl.ds(r, K)])

  return agg_kernel


def _dense_body(x_ref, wp_ref, bp_ref, wg_ref, dega_ref, degb_ref,
                mp_ref, dis_ref):
  h = jnp.maximum(
      jnp.dot(x_ref[...], wp_ref[...], preferred_element_type=jnp.float32)
      + bp_ref[...], 0.0)
  m = jnp.dot(h, wg_ref[...], preferred_element_type=jnp.float32)
  deg = dega_ref[...] + degb_ref[...] + 1.0
  dis = lax.rsqrt(deg)
  dis_ref[...] = dis
  mp_ref[...] = dis * m


def _combine_body(acc_a_ref, acc_b_ref, mp_ref, dis_ref, bg_ref, out_ref):
  out_ref[...] = (dis_ref[...]
                  * (acc_a_ref[0] + acc_b_ref[0] + mp_ref[...])
                  + bg_ref[...])


_RB = 1000


def _dense_call(x, W_proj, b_proj, W_gcn, degA, degB):
  grid = N // _RB
  row_spec = pl.BlockSpec((_RB, D), lambda i: (i, 0))
  col_spec = pl.BlockSpec((_RB, 1), lambda i: (i, 0))
  full_spec = pl.BlockSpec((D, D), lambda i: (0, 0))
  bias_spec = pl.BlockSpec((1, D), lambda i: (0, 0))
  return pl.pallas_call(
      _dense_body,
      grid=(grid,),
      in_specs=[row_spec, full_spec, bias_spec, full_spec, col_spec, col_spec],
      out_specs=[row_spec, col_spec],
      out_shape=[jax.ShapeDtypeStruct((N, D), jnp.float32),
                 jax.ShapeDtypeStruct((N, 1), jnp.float32)],
  )(x, W_proj, b_proj.reshape(1, D), W_gcn, degA, degB)


def _combine_call(acc_p, mp, dis, b_gcn):
  grid = N // _RB
  row_spec = pl.BlockSpec((_RB, D), lambda i: (i, 0))
  col_spec = pl.BlockSpec((_RB, 1), lambda i: (i, 0))
  bias_spec = pl.BlockSpec((1, D), lambda i: (0, 0))
  acc_a_spec = pl.BlockSpec((1, _RB, D), lambda i: (0, i, 0))
  acc_b_spec = pl.BlockSpec((1, _RB, D), lambda i: (1, i, 0))
  return pl.pallas_call(
      _combine_body,
      grid=(grid,),
      in_specs=[acc_a_spec, acc_b_spec, row_spec, col_spec, bias_spec],
      out_specs=row_spec,
      out_shape=jax.ShapeDtypeStruct((N, D), jnp.float32),
  )(acc_p, acc_p, mp, dis, b_gcn.reshape(1, D))


def kernel(x, edge_index, W_proj, b_proj, W_gcn, b_gcn):
  ei = edge_index.astype(jnp.int32)
  src, dst = ei[0], ei[1]

  pad = E_PAD - E
  pad_iota = jnp.arange(pad, dtype=jnp.int32)
  src_flat = jnp.concatenate([src, pad_iota % N])
  dst_flat = jnp.concatenate([dst, N + pad_iota % (N_PAD - N)])
  dst_p = dst_flat.reshape(ROWS, K)
  src_c = src_flat.reshape(CROWS, KC)
  dst_c = dst_flat.reshape(CROWS, KC)

  degA, degB = _make_deg_kernel()(dst_p)
  degA = degA[:N].reshape(N, 1)
  degB = degB[:N].reshape(N, 1)

  mp, dis = _dense_call(x, W_proj, b_proj, W_gcn, degA, degB)

  acc_p = _make_agg_kernel()(src_c, dst_c, mp)

  return _combine_call(acc_p, mp, dis, b_gcn)


# speedup vs baseline: 41.6550x; 1.1186x over previous
import functools

import jax
import jax.numpy as jnp
from jax import lax
from jax.experimental import pallas as pl
from jax.experimental.pallas import tpu as pltpu
from jax.experimental.pallas import tpu_sc as plsc

N = 10000
D = 128
E = 320000

NC = 2
NS = 16

N_PAD = 10240
E_PAD = 327680
N_PER_TILE = N_PAD // NS

KC = 128
CROWS = E_PAD // KC
CPT = CROWS // (NC * NS)
PH = 2
K = 128
ROWS = E_PAD // K
ROWS_PER_TILE = ROWS // (NC * NS)


def _make_deg_kernel():
  mesh = plsc.VectorSubcoreMesh(core_axis_name="c", subcore_axis_name="s",
                                num_cores=NC, num_subcores=NS)

  @functools.partial(
      pl.kernel,
      out_type=[jax.ShapeDtypeStruct((N_PAD,), jnp.float32),
                jax.ShapeDtypeStruct((N_PAD,), jnp.float32)],
      mesh=mesh,
      scratch_types=[
          pltpu.VMEM((ROWS_PER_TILE, K), jnp.int32),
          pltpu.VMEM((K,), jnp.float32),
          pltpu.VMEM((N_PER_TILE,), jnp.float32),
          pltpu.VMEM_SHARED((N_PAD,), jnp.float32),
      ],
  )
  def deg_kernel(dst_hbm, outa_hbm, outb_hbm, idx_v, ones_v, zeros_v, deg_sh):
    cid = lax.axis_index("c")
    sid = lax.axis_index("s")

    @pl.loop(0, N_PER_TILE // 16)
    def _zero(i):
      zeros_v[pl.ds(i * 16, 16)] = jnp.zeros((16,), jnp.float32)

    @pl.loop(0, K // 16)
    def _one(i):
      ones_v[pl.ds(i * 16, 16)] = jnp.ones((16,), jnp.float32)

    pltpu.sync_copy(zeros_v, deg_sh.at[pl.ds(sid * N_PER_TILE, N_PER_TILE)])
    plsc.subcore_barrier()

    row_base = (cid * NS + sid) * ROWS_PER_TILE
    pltpu.sync_copy(dst_hbm.at[pl.ds(row_base, ROWS_PER_TILE)], idx_v)

    @pl.loop(0, ROWS_PER_TILE)
    def _scatter(j):
      pltpu.sync_copy(ones_v, deg_sh.at[idx_v.at[j]], add=True)

    plsc.subcore_barrier()

    @pl.when(cid == 0)
    def _outa():
      pltpu.sync_copy(deg_sh.at[pl.ds(sid * N_PER_TILE, N_PER_TILE)],
                      outa_hbm.at[pl.ds(sid * N_PER_TILE, N_PER_TILE)])

    @pl.when(cid == 1)
    def _outb():
      pltpu.sync_copy(deg_sh.at[pl.ds(sid * N_PER_TILE, N_PER_TILE)],
                      outb_hbm.at[pl.ds(sid * N_PER_TILE, N_PER_TILE)])

  return deg_kernel


def _make_agg_kernel():
  mesh = plsc.VectorSubcoreMesh(core_axis_name="c", subcore_axis_name="s",
                                num_cores=NC, num_subcores=NS)

  @functools.partial(
      pl.kernel,
      out_type=jax.ShapeDtypeStruct((NC, N_PAD, D), jnp.float32),
      mesh=mesh,
      scratch_types=[
          pltpu.VMEM((CPT // PH, KC), jnp.int32),
          pltpu.VMEM((CPT // PH, KC), jnp.int32),
          pltpu.VMEM((KC, D), jnp.float32),
          pltpu.VMEM((KC, D), jnp.float32),
          pltpu.VMEM((16, D), jnp.float32),
          pltpu.VMEM_SHARED((N_PAD, D), jnp.float32),
          pltpu.SemaphoreType.DMA,
          pltpu.SemaphoreType.DMA,
      ],
  )
  def agg_kernel(src_hbm, dst_hbm, mp_hbm, out_hbm,
                 sidx_v, didx_v, rows0_v, rows1_v, zeros_v, acc_sh,
                 gsem0, gsem1):
    cid = lax.axis_index("c")
    sid = lax.axis_index("s")

    @pl.loop(0, 16)
    def _zero(r):
      for c in range(D // 16):
        zeros_v[r, pl.ds(c * 16, 16)] = jnp.zeros((16,), jnp.float32)

    @pl.loop(0, N_PER_TILE // 16)
    def _zacc(k):
      pltpu.sync_copy(zeros_v, acc_sh.at[pl.ds(sid * N_PER_TILE + k * 16, 16)])

    plsc.subcore_barrier()

    row_base = (cid * NS + sid) * CPT
    half_rows = CPT // PH
    npairs = half_rows // 2

    for half in range(PH):
      base = row_base + half * half_rows
      pltpu.sync_copy(src_hbm.at[pl.ds(base, half_rows)], sidx_v)
      pltpu.sync_copy(dst_hbm.at[pl.ds(base, half_rows)], didx_v)

      pltpu.async_copy(mp_hbm.at[sidx_v.at[0]], rows0_v, gsem0)
      pltpu.async_copy(mp_hbm.at[sidx_v.at[1]], rows1_v, gsem1)

      @pl.loop(0, npairs)
      def _pair(p):
        c0 = 2 * p
        pltpu.make_async_copy(mp_hbm.at[sidx_v.at[0]], rows0_v, gsem0).wait()
        pltpu.sync_copy(rows0_v, acc_sh.at[didx_v.at[c0]], add=True)

        @pl.when(p < npairs - 1)
        def _g0():
          pltpu.async_copy(mp_hbm.at[sidx_v.at[c0 + 2]], rows0_v, gsem0)

        pltpu.make_async_copy(mp_hbm.at[sidx_v.at[1]], rows1_v, gsem1).wait()
        pltpu.sync_copy(rows1_v, acc_sh.at[didx_v.at[c0 + 1]], add=True)

        @pl.when(p < npairs - 1)
        def _g1():
          pltpu.async_copy(mp_hbm.at[sidx_v.at[c0 + 3]], rows1_v, gsem1)

    plsc.subcore_barrier()

    @pl.loop(0, N_PER_TILE // K)
    def _out(k):
      r = sid * N_PER_TILE + k * K
      pltpu.sync_copy(acc_sh.at[pl.ds(r, K)], out_hbm.at[cid, pl.ds(r, K)])

  return agg_kernel


def _dense_body(x_ref, wp_ref, bp_ref, wg_ref, dega_ref, degb_ref,
                mp_ref, dis_ref):
  h = jnp.maximum(
      jnp.dot(x_ref[...], wp_ref[...], preferred_element_type=jnp.float32)
      + bp_ref[...], 0.0)
  m = jnp.dot(h, wg_ref[...], preferred_element_type=jnp.float32)
  deg = dega_ref[...] + degb_ref[...] + 1.0
  dis = lax.rsqrt(deg)
  dis_ref[...] = dis
  mp_ref[...] = dis * m


def _combine_body(acc_a_ref, acc_b_ref, mp_ref, dis_ref, bg_ref, out_ref):
  out_ref[...] = (dis_ref[...]
                  * (acc_a_ref[0] + acc_b_ref[0] + mp_ref[...])
                  + bg_ref[...])


_RB = 1000


def _dense_call(x, W_proj, b_proj, W_gcn, degA, degB):
  grid = N // _RB
  row_spec = pl.BlockSpec((_RB, D), lambda i: (i, 0))
  col_spec = pl.BlockSpec((_RB, 1), lambda i: (i, 0))
  full_spec = pl.BlockSpec((D, D), lambda i: (0, 0))
  bias_spec = pl.BlockSpec((1, D), lambda i: (0, 0))
  return pl.pallas_call(
      _dense_body,
      grid=(grid,),
      in_specs=[row_spec, full_spec, bias_spec, full_spec, col_spec, col_spec],
      out_specs=[row_spec, col_spec],
      out_shape=[jax.ShapeDtypeStruct((N, D), jnp.float32),
                 jax.ShapeDtypeStruct((N, 1), jnp.float32)],
  )(x, W_proj, b_proj.reshape(1, D), W_gcn, degA, degB)


def _combine_call(acc_p, mp, dis, b_gcn):
  grid = N // _RB
  row_spec = pl.BlockSpec((_RB, D), lambda i: (i, 0))
  col_spec = pl.BlockSpec((_RB, 1), lambda i: (i, 0))
  bias_spec = pl.BlockSpec((1, D), lambda i: (0, 0))
  acc_a_spec = pl.BlockSpec((1, _RB, D), lambda i: (0, i, 0))
  acc_b_spec = pl.BlockSpec((1, _RB, D), lambda i: (1, i, 0))
  return pl.pallas_call(
      _combine_body,
      grid=(grid,),
      in_specs=[acc_a_spec, acc_b_spec, row_spec, col_spec, bias_spec],
      out_specs=row_spec,
      out_shape=jax.ShapeDtypeStruct((N, D), jnp.float32),
  )(acc_p, acc_p, mp, dis, b_gcn.reshape(1, D))


def kernel(x, edge_index, W_proj, b_proj, W_gcn, b_gcn):
  ei = edge_index.astype(jnp.int32)
  src, dst = ei[0], ei[1]

  pad = E_PAD - E
  pad_iota = jnp.arange(pad, dtype=jnp.int32)
  src_flat = jnp.concatenate([src, pad_iota % N])
  dst_flat = jnp.concatenate([dst, N + pad_iota % (N_PAD - N)])
  dst_p = dst_flat.reshape(ROWS, K)
  src_c = src_flat.reshape(CROWS, KC)
  dst_c = dst_flat.reshape(CROWS, KC)

  degA, degB = _make_deg_kernel()(dst_p)
  degA = degA[:N].reshape(N, 1)
  degB = degB[:N].reshape(N, 1)

  mp, dis = _dense_call(x, W_proj, b_proj, W_gcn, degA, degB)

  acc_p = _make_agg_kernel()(src_c, dst_c, mp)

  return _combine_call(acc_p, mp, dis, b_gcn)


# speedup vs baseline: 42.9138x; 1.0302x over previous
import functools

import jax
import jax.numpy as jnp
from jax import lax
from jax.experimental import pallas as pl
from jax.experimental.pallas import tpu as pltpu
from jax.experimental.pallas import tpu_sc as plsc

N = 10000
D = 128
E = 320000

NC = 2
NS = 16

N_PAD = 10240
E_PAD = 327680
N_PER_TILE = N_PAD // NS

KC = 128
CROWS = E_PAD // KC
CPT = CROWS // (NC * NS)
PH = 2
K = 128
ROWS = E_PAD // K
ROWS_PER_TILE = ROWS // (NC * NS)


def _make_deg_kernel():
  mesh = plsc.VectorSubcoreMesh(core_axis_name="c", subcore_axis_name="s",
                                num_cores=NC, num_subcores=NS)

  @functools.partial(
      pl.kernel,
      out_type=[jax.ShapeDtypeStruct((N_PAD,), jnp.float32),
                jax.ShapeDtypeStruct((N_PAD,), jnp.float32)],
      mesh=mesh,
      scratch_types=[
          pltpu.VMEM((ROWS_PER_TILE, K), jnp.int32),
          pltpu.VMEM((K,), jnp.float32),
          pltpu.VMEM((N_PER_TILE,), jnp.float32),
          pltpu.VMEM_SHARED((N_PAD,), jnp.float32),
      ],
  )
  def deg_kernel(dst_hbm, outa_hbm, outb_hbm, idx_v, ones_v, zeros_v, deg_sh):
    cid = lax.axis_index("c")
    sid = lax.axis_index("s")

    @pl.loop(0, N_PER_TILE // 16)
    def _zero(i):
      zeros_v[pl.ds(i * 16, 16)] = jnp.zeros((16,), jnp.float32)

    @pl.loop(0, K // 16)
    def _one(i):
      ones_v[pl.ds(i * 16, 16)] = jnp.ones((16,), jnp.float32)

    pltpu.sync_copy(zeros_v, deg_sh.at[pl.ds(sid * N_PER_TILE, N_PER_TILE)])
    plsc.subcore_barrier()

    row_base = (cid * NS + sid) * ROWS_PER_TILE
    pltpu.sync_copy(dst_hbm.at[pl.ds(row_base, ROWS_PER_TILE)], idx_v)

    @pl.loop(0, ROWS_PER_TILE)
    def _scatter(j):
      pltpu.sync_copy(ones_v, deg_sh.at[idx_v.at[j]], add=True)

    plsc.subcore_barrier()

    @pl.when(cid == 0)
    def _outa():
      pltpu.sync_copy(deg_sh.at[pl.ds(sid * N_PER_TILE, N_PER_TILE)],
                      outa_hbm.at[pl.ds(sid * N_PER_TILE, N_PER_TILE)])

    @pl.when(cid == 1)
    def _outb():
      pltpu.sync_copy(deg_sh.at[pl.ds(sid * N_PER_TILE, N_PER_TILE)],
                      outb_hbm.at[pl.ds(sid * N_PER_TILE, N_PER_TILE)])

  return deg_kernel


def _make_agg_kernel():
  mesh = plsc.VectorSubcoreMesh(core_axis_name="c", subcore_axis_name="s",
                                num_cores=NC, num_subcores=NS)

  @functools.partial(
      pl.kernel,
      out_type=jax.ShapeDtypeStruct((NC, N_PAD, D), jnp.float32),
      mesh=mesh,
      scratch_types=[
          pltpu.VMEM((CPT // PH, KC), jnp.int32),
          pltpu.VMEM((CPT // PH, KC), jnp.int32),
          pltpu.VMEM((KC, D), jnp.float32),
          pltpu.VMEM((KC, D), jnp.float32),
          pltpu.VMEM((16, D), jnp.float32),
          pltpu.VMEM_SHARED((N_PAD, D), jnp.float32),
          pltpu.SemaphoreType.DMA,
          pltpu.SemaphoreType.DMA,
      ],
  )
  def agg_kernel(src_hbm, dst_hbm, mp_hbm, out_hbm,
                 sidx_v, didx_v, rows0_v, rows1_v, zeros_v, acc_sh,
                 gsem0, gsem1):
    cid = lax.axis_index("c")
    sid = lax.axis_index("s")

    @pl.loop(0, 16)
    def _zero(r):
      for c in range(D // 16):
        zeros_v[r, pl.ds(c * 16, 16)] = jnp.zeros((16,), jnp.float32)

    @pl.loop(0, N_PER_TILE // 16)
    def _zacc(k):
      pltpu.sync_copy(zeros_v, acc_sh.at[pl.ds(sid * N_PER_TILE + k * 16, 16)])

    plsc.subcore_barrier()

    row_base = (cid * NS + sid) * CPT
    half_rows = CPT // PH
    npairs = half_rows // 2

    for half in range(PH):
      base = row_base + half * half_rows
      pltpu.sync_copy(src_hbm.at[pl.ds(base, half_rows)], sidx_v)
      pltpu.sync_copy(dst_hbm.at[pl.ds(base, half_rows)], didx_v)

      pltpu.async_copy(mp_hbm.at[sidx_v.at[0]], rows0_v, gsem0)
      pltpu.async_copy(mp_hbm.at[sidx_v.at[1]], rows1_v, gsem1)

      @pl.loop(0, npairs)
      def _pair(p):
        c0 = 2 * p
        pltpu.make_async_copy(mp_hbm.at[sidx_v.at[0]], rows0_v, gsem0).wait()
        pltpu.sync_copy(rows0_v, acc_sh.at[didx_v.at[c0]], add=True)

        @pl.when(p < npairs - 1)
        def _g0():
          pltpu.async_copy(mp_hbm.at[sidx_v.at[c0 + 2]], rows0_v, gsem0)

        pltpu.make_async_copy(mp_hbm.at[sidx_v.at[1]], rows1_v, gsem1).wait()
        pltpu.sync_copy(rows1_v, acc_sh.at[didx_v.at[c0 + 1]], add=True)

        @pl.when(p < npairs - 1)
        def _g1():
          pltpu.async_copy(mp_hbm.at[sidx_v.at[c0 + 3]], rows1_v, gsem1)

    plsc.subcore_barrier()

    @pl.loop(0, N_PER_TILE // K)
    def _out(k):
      r = sid * N_PER_TILE + k * K
      pltpu.sync_copy(acc_sh.at[pl.ds(r, K)], out_hbm.at[cid, pl.ds(r, K)])

  return agg_kernel


def _matmul_body(x_ref, wp_ref, bp_ref, wg_ref, m_ref):
  h = jnp.maximum(
      jnp.dot(x_ref[...], wp_ref[...], preferred_element_type=jnp.float32)
      + bp_ref[...], 0.0)
  m_ref[...] = jnp.dot(h, wg_ref[...], preferred_element_type=jnp.float32)


def _scale_body(m_ref, dega_ref, degb_ref, mp_ref, dis_ref):
  deg = dega_ref[...] + degb_ref[...] + 1.0
  dis = lax.rsqrt(deg)
  dis_ref[...] = dis
  mp_ref[...] = dis * m_ref[...]


def _combine_body(acc_a_ref, acc_b_ref, mp_ref, dis_ref, bg_ref, out_ref):
  out_ref[...] = (dis_ref[...]
                  * (acc_a_ref[0] + acc_b_ref[0] + mp_ref[...])
                  + bg_ref[...])


_RB = 2000


def _matmul_call(x, W_proj, b_proj, W_gcn):
  grid = N // _RB
  row_spec = pl.BlockSpec((_RB, D), lambda i: (i, 0))
  full_spec = pl.BlockSpec((D, D), lambda i: (0, 0))
  bias_spec = pl.BlockSpec((1, D), lambda i: (0, 0))
  return pl.pallas_call(
      _matmul_body,
      grid=(grid,),
      in_specs=[row_spec, full_spec, bias_spec, full_spec],
      out_specs=row_spec,
      out_shape=jax.ShapeDtypeStruct((N, D), jnp.float32),
  )(x, W_proj, b_proj.reshape(1, D), W_gcn)


def _scale_call(m, degA, degB):
  grid = N // _RB
  row_spec = pl.BlockSpec((_RB, D), lambda i: (i, 0))
  col_spec = pl.BlockSpec((_RB, 1), lambda i: (i, 0))
  return pl.pallas_call(
      _scale_body,
      grid=(grid,),
      in_specs=[row_spec, col_spec, col_spec],
      out_specs=[row_spec, col_spec],
      out_shape=[jax.ShapeDtypeStruct((N, D), jnp.float32),
                 jax.ShapeDtypeStruct((N, 1), jnp.float32)],
  )(m, degA, degB)


def _combine_call(acc_p, mp, dis, b_gcn):
  grid = N // _RB
  row_spec = pl.BlockSpec((_RB, D), lambda i: (i, 0))
  col_spec = pl.BlockSpec((_RB, 1), lambda i: (i, 0))
  bias_spec = pl.BlockSpec((1, D), lambda i: (0, 0))
  acc_a_spec = pl.BlockSpec((1, _RB, D), lambda i: (0, i, 0))
  acc_b_spec = pl.BlockSpec((1, _RB, D), lambda i: (1, i, 0))
  return pl.pallas_call(
      _combine_body,
      grid=(grid,),
      in_specs=[acc_a_spec, acc_b_spec, row_spec, col_spec, bias_spec],
      out_specs=row_spec,
      out_shape=jax.ShapeDtypeStruct((N, D), jnp.float32),
  )(acc_p, acc_p, mp, dis, b_gcn.reshape(1, D))


def kernel(x, edge_index, W_proj, b_proj, W_gcn, b_gcn):
  ei = edge_index.astype(jnp.int32)
  src, dst = ei[0], ei[1]

  pad = E_PAD - E
  pad_iota = jnp.arange(pad, dtype=jnp.int32)
  src_flat = jnp.concatenate([src, pad_iota % N])
  dst_flat = jnp.concatenate([dst, N + pad_iota % (N_PAD - N)])
  src_c = src_flat.reshape(CROWS, KC)
  dst_c = dst_flat.reshape(CROWS, KC)

  m = _matmul_call(x, W_proj, b_proj, W_gcn)

  degA, degB = _make_deg_kernel()(dst_c)
  degA = degA[:N].reshape(N, 1)
  degB = degB[:N].reshape(N, 1)

  mp, dis = _scale_call(m, degA, degB)

  acc_p = _make_agg_kernel()(src_c, dst_c, mp)

  return _combine_call(acc_p, mp, dis, b_gcn)


# speedup vs baseline: 47.8089x; 1.1141x over previous
import functools

import jax
import jax.numpy as jnp
from jax import lax
from jax.experimental import pallas as pl
from jax.experimental.pallas import tpu as pltpu
from jax.experimental.pallas import tpu_sc as plsc

N = 10000
D = 128
E = 320000

NC = 2
NS = 16

N_PAD = 10240
E_PAD = 327680
N_PER_TILE = N_PAD // NS

KC = 128
CROWS = E_PAD // KC
CPT = CROWS // (NC * NS)
PH = 2
K = 128
ROWS = E_PAD // K
ROWS_PER_TILE = ROWS // (NC * NS)


def _make_deg_kernel():
  mesh = plsc.VectorSubcoreMesh(core_axis_name="c", subcore_axis_name="s",
                                num_cores=NC, num_subcores=NS)

  @functools.partial(
      pl.kernel,
      out_type=jax.ShapeDtypeStruct((NC, N_PAD, 16), jnp.float32),
      mesh=mesh,
      scratch_types=[
          pltpu.VMEM((ROWS_PER_TILE, K), jnp.int32),
          pltpu.VMEM((K,), jnp.float32),
          pltpu.VMEM((N_PER_TILE + 16,), jnp.float32),
          pltpu.VMEM((N_PER_TILE, 16), jnp.float32),
          pltpu.VMEM_SHARED((N_PAD,), jnp.float32),
          pltpu.SemaphoreType.DMA,
      ],
  )
  def deg_kernel(ei_hbm, out_hbm, idx_v, ones_v, zeros_v, stage_v, deg_sh,
                 dsem):
    cid = lax.axis_index("c")
    sid = lax.axis_index("s")

    @pl.loop(0, N_PER_TILE // 16 + 1)
    def _zero(i):
      zeros_v[pl.ds(i * 16, 16)] = jnp.zeros((16,), jnp.float32)

    @pl.loop(0, K // 16)
    def _one(i):
      ones_v[pl.ds(i * 16, 16)] = jnp.ones((16,), jnp.float32)

    pltpu.sync_copy(zeros_v.at[pl.ds(0, N_PER_TILE)],
                    deg_sh.at[pl.ds(sid * N_PER_TILE, N_PER_TILE)])
    plsc.subcore_barrier()

    row_base = (cid * NS + sid) * ROWS_PER_TILE
    pltpu.sync_copy(ei_hbm.at[1, pl.ds(row_base, ROWS_PER_TILE)], idx_v)

    @pl.loop(0, ROWS_PER_TILE // 16)
    def _scatter(b):
      for j in range(16):
        pltpu.async_copy(ones_v, deg_sh.at[idx_v.at[b * 16 + j]], dsem,
                         add=True)
      for j in range(16):
        pltpu.make_async_copy(ones_v, deg_sh.at[idx_v.at[0]], dsem).wait()

    plsc.subcore_barrier()

    base = sid * N_PER_TILE
    pltpu.sync_copy(deg_sh.at[pl.ds(base, N_PER_TILE)],
                    zeros_v.at[pl.ds(0, N_PER_TILE)])

    @pl.loop(0, N_PER_TILE)
    def _w(r):
      stage_v[r, pl.ds(0, 16)] = zeros_v[pl.ds(r, 16)]

    pltpu.sync_copy(stage_v, out_hbm.at[cid, pl.ds(base, N_PER_TILE)])

  return deg_kernel


def _make_agg_kernel():
  mesh = plsc.VectorSubcoreMesh(core_axis_name="c", subcore_axis_name="s",
                                num_cores=NC, num_subcores=NS)

  @functools.partial(
      pl.kernel,
      out_type=jax.ShapeDtypeStruct((NC, N_PAD, D), jnp.float32),
      mesh=mesh,
      scratch_types=[
          pltpu.VMEM((CPT // PH, KC), jnp.int32),
          pltpu.VMEM((CPT // PH, KC), jnp.int32),
          pltpu.VMEM((KC, D), jnp.float32),
          pltpu.VMEM((KC, D), jnp.float32),
          pltpu.VMEM((16, D), jnp.float32),
          pltpu.VMEM_SHARED((N_PAD, D), jnp.float32),
          pltpu.SemaphoreType.DMA,
          pltpu.SemaphoreType.DMA,
          pltpu.SemaphoreType.DMA,
      ],
  )
  def agg_kernel(ei_hbm, mp_hbm, out_hbm,
                 sidx_v, didx_v, rows0_v, rows1_v, zeros_v, acc_sh,
                 gsem0, gsem1, zsem):
    cid = lax.axis_index("c")
    sid = lax.axis_index("s")

    @pl.loop(0, 16)
    def _zero(r):
      for c in range(D // 16):
        zeros_v[r, pl.ds(c * 16, 16)] = jnp.zeros((16,), jnp.float32)

    @pl.loop(0, N_PER_TILE // 16 // 8)
    def _zacc(k):
      for j in range(8):
        r = sid * N_PER_TILE + (k * 8 + j) * 16
        pltpu.async_copy(zeros_v, acc_sh.at[pl.ds(r, 16)], zsem)
      for j in range(8):
        pltpu.make_async_copy(zeros_v, acc_sh.at[pl.ds(0, 16)], zsem).wait()

    plsc.subcore_barrier()

    row_base = (cid * NS + sid) * CPT
    half_rows = CPT // PH
    npairs = half_rows // 2

    for half in range(PH):
      base = row_base + half * half_rows
      pltpu.sync_copy(ei_hbm.at[0, pl.ds(base, half_rows)], sidx_v)
      pltpu.sync_copy(ei_hbm.at[1, pl.ds(base, half_rows)], didx_v)

      pltpu.async_copy(mp_hbm.at[sidx_v.at[0]], rows0_v, gsem0)
      pltpu.async_copy(mp_hbm.at[sidx_v.at[1]], rows1_v, gsem1)

      @pl.loop(0, npairs)
      def _pair(p):
        c0 = 2 * p
        pltpu.make_async_copy(mp_hbm.at[sidx_v.at[0]], rows0_v, gsem0).wait()
        pltpu.sync_copy(rows0_v, acc_sh.at[didx_v.at[c0]], add=True)

        @pl.when(p < npairs - 1)
        def _g0():
          pltpu.async_copy(mp_hbm.at[sidx_v.at[c0 + 2]], rows0_v, gsem0)

        pltpu.make_async_copy(mp_hbm.at[sidx_v.at[1]], rows1_v, gsem1).wait()
        pltpu.sync_copy(rows1_v, acc_sh.at[didx_v.at[c0 + 1]], add=True)

        @pl.when(p < npairs - 1)
        def _g1():
          pltpu.async_copy(mp_hbm.at[sidx_v.at[c0 + 3]], rows1_v, gsem1)

    plsc.subcore_barrier()

    for k in range(N_PER_TILE // K):
      r = sid * N_PER_TILE + k * K
      pltpu.async_copy(acc_sh.at[pl.ds(r, K)], out_hbm.at[cid, pl.ds(r, K)],
                       zsem)
    for k in range(N_PER_TILE // K):
      r = sid * N_PER_TILE + k * K
      pltpu.make_async_copy(acc_sh.at[pl.ds(r, K)],
                            out_hbm.at[cid, pl.ds(r, K)], zsem).wait()

  return agg_kernel


def _matmul_body(x_ref, wp_ref, bp_ref, wg_ref, m_ref):
  h = jnp.maximum(
      jnp.dot(x_ref[...], wp_ref[...], preferred_element_type=jnp.float32)
      + bp_ref[...], 0.0)
  m_ref[...] = jnp.dot(h, wg_ref[...], preferred_element_type=jnp.float32)


def _scale_body(m_ref, dega_ref, degb_ref, mp_ref, dis_ref):
  deg = dega_ref[0][:, 0:1] + degb_ref[0][:, 0:1] + 1.0
  dis = lax.rsqrt(deg)
  dis_ref[...] = dis
  mp_ref[...] = dis * m_ref[...]


def _combine_body(acc_a_ref, acc_b_ref, mp_ref, dis_ref, bg_ref, out_ref):
  out_ref[...] = (dis_ref[...]
                  * (acc_a_ref[0] + acc_b_ref[0] + mp_ref[...])
                  + bg_ref[...])


_RB = 5000


def _matmul_call(x, W_proj, b_proj, W_gcn):
  grid = N // _RB
  row_spec = pl.BlockSpec((_RB, D), lambda i: (i, 0))
  full_spec = pl.BlockSpec((D, D), lambda i: (0, 0))
  bias_spec = pl.BlockSpec((1, D), lambda i: (0, 0))
  return pl.pallas_call(
      _matmul_body,
      grid=(grid,),
      in_specs=[row_spec, full_spec, bias_spec, full_spec],
      out_specs=row_spec,
      out_shape=jax.ShapeDtypeStruct((N, D), jnp.float32),
  )(x, W_proj, b_proj.reshape(1, D), W_gcn)


def _scale_call(m, deg_p):
  grid = N // _RB
  row_spec = pl.BlockSpec((_RB, D), lambda i: (i, 0))
  col_spec = pl.BlockSpec((_RB, 1), lambda i: (i, 0))
  dega_spec = pl.BlockSpec((1, _RB, 16), lambda i: (0, i, 0))
  degb_spec = pl.BlockSpec((1, _RB, 16), lambda i: (1, i, 0))
  return pl.pallas_call(
      _scale_body,
      grid=(grid,),
      in_specs=[row_spec, dega_spec, degb_spec],
      out_specs=[row_spec, col_spec],
      out_shape=[jax.ShapeDtypeStruct((N, D), jnp.float32),
                 jax.ShapeDtypeStruct((N, 1), jnp.float32)],
  )(m, deg_p, deg_p)


def _combine_call(acc_p, mp, dis, b_gcn):
  grid = N // _RB
  row_spec = pl.BlockSpec((_RB, D), lambda i: (i, 0))
  col_spec = pl.BlockSpec((_RB, 1), lambda i: (i, 0))
  bias_spec = pl.BlockSpec((1, D), lambda i: (0, 0))
  acc_a_spec = pl.BlockSpec((1, _RB, D), lambda i: (0, i, 0))
  acc_b_spec = pl.BlockSpec((1, _RB, D), lambda i: (1, i, 0))
  return pl.pallas_call(
      _combine_body,
      grid=(grid,),
      in_specs=[acc_a_spec, acc_b_spec, row_spec, col_spec, bias_spec],
      out_specs=row_spec,
      out_shape=jax.ShapeDtypeStruct((N, D), jnp.float32),
  )(acc_p, acc_p, mp, dis, b_gcn.reshape(1, D))


def kernel(x, edge_index, W_proj, b_proj, W_gcn, b_gcn):
  ei = edge_index.astype(jnp.int32)

  pad = E_PAD - E
  pad_iota = jnp.arange(pad, dtype=jnp.int32)
  pad2 = jnp.stack([pad_iota % N, N + pad_iota % (N_PAD - N)])
  ei3 = jnp.concatenate([ei, pad2], axis=1).reshape(2, CROWS, KC)

  m = _matmul_call(x, W_proj, b_proj, W_gcn)

  deg_p = _make_deg_kernel()(ei3)

  mp, dis = _scale_call(m, deg_p)

  acc_p = _make_agg_kernel()(ei3, mp)

  return _combine_call(acc_p, mp, dis, b_gcn)
